# baseline (speedup 1.0000x reference)
import jax
import jax.numpy as jnp
from jax import lax
from jax.experimental import pallas as pl
from jax.experimental.pallas import tpu as pltpu
from jax.experimental.pallas import tpu_sc as plsc

_N = 10000
_E = 80000
_R = 4
_D = 128
_NB = 2

_NC = 2
_NS = 16
_LANES = 16

_CHUNK = 48
_NCHK = 432
_EPT = _NCHK * _CHUNK
_EPAD = (_NS // _R) * _EPT
_NPAD = 10240
_ROWS_PT = _NPAD // _NS
_UNROLL = 6


def _sc_body(inp_hbm, src_hbm, dst_hbm, val_hbm, coeff_hbm, acc_hbm,
             acc_sp, val_b,
             sx0, sx1, sx2, sx3, sx4, sx5,
             dx0, dx1, dx2, dx3, dx4, dx5,
             gb0, gb1, gb2, sb0, sb1, coeff_v,
             gs0, gs1, gs2, ss0, ss1, is0, is1, is2):
    c = lax.axis_index("c")
    s = lax.axis_index("s")

    gbufs = (gb0, gb1, gb2)
    sbufs = (sb0, sb1)
    srcb = (sx0, sx1, sx2, sx3, sx4, sx5)
    dstb = (dx0, dx1, dx2, dx3, dx4, dx5)
    gsems = (gs0, gs1, gs2)
    ssems = (ss0, ss1)
    isems = (is0, is1, is2)

    pltpu.sync_copy(coeff_hbm, coeff_v)

    def _zero(buf):
        def _zr(i, carry):
            for j in range(_D // _LANES):
                buf[i, pl.ds(j * _LANES, _LANES)] = jnp.zeros((_LANES,), jnp.float32)
            return carry
        lax.fori_loop(0, _CHUNK, _zr, 0)
    _zero(gb0)
    _zero(sb0)
    _zero(sb1)
    for t in range(_ROWS_PT // _CHUNK):
        pltpu.sync_copy(gb0, acc_sp.at[pl.ds(s * _ROWS_PT + t * _CHUNK, _CHUNK)])
    _ZREM = _ROWS_PT - (_ROWS_PT // _CHUNK) * _CHUNK
    if _ZREM:
        pltpu.sync_copy(
            gb0.at[pl.ds(0, _ZREM)],
            acc_sp.at[pl.ds(s * _ROWS_PT + (_ROWS_PT // _CHUNK) * _CHUNK, _ZREM)])

    rel = s // (_NS // _R)
    cv = coeff_v[...]
    want = jnp.full((_LANES,), rel * _NB + c, jnp.int32)
    cvec = cv.at[want].get(mode="promise_in_bounds")

    plsc.subcore_barrier()

    def _ld_idx(k, row, sem):
        pltpu.async_copy(src_hbm.at[s, k], srcb[row], sem)
        pltpu.async_copy(dst_hbm.at[s, k], dstb[row], sem)
        pltpu.async_copy(val_hbm.at[s, k], val_b.at[row], sem)

    def _wait_idx(sem):
        pltpu.make_async_copy(src_hbm.at[s, 0], srcb[0], sem).wait()
        pltpu.make_async_copy(dst_hbm.at[s, 0], dstb[0], sem).wait()
        pltpu.make_async_copy(val_hbm.at[s, 0], val_b.at[0], sem).wait()

    def _gather(row, buf, sem):
        pltpu.async_copy(inp_hbm.at[srcb[row]], buf, sem)

    def _wait_g(buf, sem):
        pltpu.make_async_copy(inp_hbm.at[srcb[0]], buf, sem).wait()

    def _scatter(row, buf, sem):
        pltpu.async_copy(buf, acc_sp.at[dstb[row]], sem, add=True)

    def _wait_s(buf, sem):
        pltpu.make_async_copy(buf, acc_sp.at[dstb[0]], sem).wait()

    def _scale(gbuf, sbuf, row):
        def _gg(g, carry):
            w16 = val_b[row, pl.ds(g * _LANES, _LANES)] * cvec
            for l in range(_LANES):
                w = w16[l]
                e = g * _LANES + l
                for j in range(_D // _LANES):
                    sl = pl.ds(j * _LANES, _LANES)
                    sbuf[e, sl] = gbuf[e, sl] * w
            return carry
        lax.fori_loop(0, _CHUNK // _LANES, _gg, 0)

    for u in range(3):
        pltpu.sync_copy(src_hbm.at[s, u], srcb[u])
        pltpu.sync_copy(dst_hbm.at[s, u], dstb[u])
        pltpu.sync_copy(val_hbm.at[s, u], val_b.at[u])
    for u in range(3):
        _gather(u, gbufs[u], gsems[u])
    _ld_idx(jnp.int32(3), 3, isems[2])
    _scatter(0, sb0, ss0)
    _scatter(0, sb1, ss1)

    def _body(m, carry):
        k0 = _UNROLL * m
        for u in range(_UNROLL):
            k = k0 + u
            gi, si, ii = u % 3, u % 2, u % 3
            _wait_g(gbufs[gi], gsems[gi])
            _wait_s(sbufs[si], ssems[si])
            _ld_idx(jnp.minimum(k + 4, _NCHK - 1), (u + 4) % _UNROLL,
                    isems[ii])
            _scale(gbufs[gi], sbufs[si], u)
            _scatter(u, sbufs[si], ssems[si])
            _wait_idx(isems[(u + 2) % 3])
            _gather((u + 3) % _UNROLL, gbufs[gi], gsems[gi])
        return carry
    lax.fori_loop(0, _NCHK // _UNROLL, _body, 0)

    for u in range(3):
        _wait_g(gbufs[u], gsems[u])
    _wait_idx(isems[(_NCHK - 1) % 3])
    _wait_s(sb0, ss0)
    _wait_s(sb1, ss1)

    plsc.subcore_barrier()
    rsl = pl.ds(s * _ROWS_PT, _ROWS_PT)
    pltpu.sync_copy(acc_sp.at[rsl], acc_hbm.at[c, rsl])


@jax.jit
def _sc_call(inp, src, dst, val, coeff_flat):
    mesh = plsc.VectorSubcoreMesh(core_axis_name="c", subcore_axis_name="s",
                                  num_cores=_NC, num_subcores=_NS)
    return pl.kernel(
        _sc_body,
        out_type=jax.ShapeDtypeStruct((_NB, _NPAD, _D), jnp.float32),
        mesh=mesh,
        scratch_types=[
            pltpu.VMEM_SHARED((_NPAD, _D), jnp.float32),
            pltpu.VMEM((_UNROLL, _CHUNK), jnp.float32),
        ] + [pltpu.VMEM((_CHUNK,), jnp.int32) for _ in range(2 * _UNROLL)] + [
            pltpu.VMEM((_CHUNK, _D), jnp.float32),
            pltpu.VMEM((_CHUNK, _D), jnp.float32),
            pltpu.VMEM((_CHUNK, _D), jnp.float32),
            pltpu.VMEM((_CHUNK, _D), jnp.float32),
            pltpu.VMEM((_CHUNK, _D), jnp.float32),
            pltpu.VMEM((_LANES,), jnp.float32),
            pltpu.SemaphoreType.DMA,
            pltpu.SemaphoreType.DMA,
            pltpu.SemaphoreType.DMA,
            pltpu.SemaphoreType.DMA,
            pltpu.SemaphoreType.DMA,
            pltpu.SemaphoreType.DMA,
            pltpu.SemaphoreType.DMA,
            pltpu.SemaphoreType.DMA,
        ],
    )(inp, src, dst, val, coeff_flat)


_BLK = 2000


def _tc_body(acc_ref, bw_ref, bias_ref, out_ref):
    a0 = acc_ref[0]
    a1 = acc_ref[1]
    out = jnp.dot(a0, bw_ref[0], preferred_element_type=jnp.float32)
    out = out + jnp.dot(a1, bw_ref[1], preferred_element_type=jnp.float32)
    out_ref[...] = out + jnp.sum(bias_ref[...], axis=0)[None, :]


@jax.jit
def _tc_call(acc, basis_weights, bias):
    return pl.pallas_call(
        _tc_body,
        out_shape=jax.ShapeDtypeStruct((_N, _D), jnp.float32),
        grid=(_N // _BLK,),
        in_specs=[
            pl.BlockSpec((_NB, _BLK, _D), lambda i: (0, i, 0)),
            pl.BlockSpec((_NB, _D, _D), lambda i: (0, 0, 0)),
            pl.BlockSpec((_R, _D), lambda i: (0, 0)),
        ],
        out_specs=pl.BlockSpec((_BLK, _D), lambda i: (i, 0)),
    )(acc, basis_weights, bias)


def _edges3(x):
    xp = jnp.pad(x, ((0, 0), (0, _EPAD - _E)))
    return xp.reshape(_NS, _NCHK, _CHUNK)


def kernel(inp, edge_index, edge_val, basis_weights, basis_coeff, bias):
    dst = _edges3(edge_index[:, 0, :])
    src = _edges3(edge_index[:, 1, :])
    val = _edges3(edge_val)
    coeff_flat = jnp.zeros((_LANES,), jnp.float32).at[: _R * _NB].set(
        basis_coeff.reshape(-1))
    acc = _sc_call(inp, src, dst, val, coeff_flat)
    return _tc_call(acc, basis_weights, bias)


# speedup vs baseline: 1.2600x; 1.2600x over previous
import jax
import jax.numpy as jnp
import numpy as np
from jax import lax
from jax.experimental import pallas as pl
from jax.experimental.pallas import tpu as pltpu
from jax.experimental.pallas import tpu_sc as plsc

_N = 10000
_E = 80000
_R = 4
_D = 128
_DW = _D // 2
_NB = 2

_NC = 2
_NS = 16
_LANES = 16

_GCH = 128
_SCH = 64
_BLKE = 2 * _GCH
_NBLK = 81
_EPT = _NBLK * _BLKE
_EPAD = (_NS // _R) * _EPT
_NPAD = 10240
_ROWS_PT = _NPAD // _NS


def _sc_body(inp_hbm, src_hbm, dst_hbm, val_hbm, coeff_hbm, acc_hbm,
             acc_sp, srcB, dstB, valB, sxa, sxb, sxc, dxa, dxb,
             gb0, gb1, gb2, sb0, sb1, coeff_v,
             gs0, gs1, gs2, ss0, ss1, bsem):
    c = lax.axis_index("c")
    s = lax.axis_index("s")

    gbufs = (gb0, gb1, gb2)
    sbufs = (sb0, sb1)
    gsems = (gs0, gs1, gs2)
    ssems = (ss0, ss1)
    dstb = (dxa, dxb)
    srcw = (sxa, sxb, sxc)

    pltpu.sync_copy(coeff_hbm, coeff_v)

    def _zero(buf):
        def _zr(i, carry):
            for j in range(_D // _LANES):
                buf[i, pl.ds(j * _LANES, _LANES)] = jnp.zeros((_LANES,), jnp.float32)
            return carry
        lax.fori_loop(0, _SCH, _zr, 0)
    _zero(sb0)
    _zero(sb1)
    for t in range(_SCH // _LANES):
        dxa[pl.ds(t * _LANES, _LANES)] = jnp.zeros((_LANES,), jnp.int32)
        dxb[pl.ds(t * _LANES, _LANES)] = jnp.zeros((_LANES,), jnp.int32)
    for t in range(_ROWS_PT // _SCH):
        pltpu.sync_copy(sb0, acc_sp.at[pl.ds(s * _ROWS_PT + t * _SCH, _SCH)])
    _ZREM = _ROWS_PT - (_ROWS_PT // _SCH) * _SCH
    if _ZREM:
        pltpu.sync_copy(
            sb0.at[pl.ds(0, _ZREM)],
            acc_sp.at[pl.ds(s * _ROWS_PT + (_ROWS_PT // _SCH) * _SCH, _ZREM)])

    rel = s // (_NS // _R)
    cv = coeff_v[...]
    want = jnp.full((_LANES,), rel * _NB + c, jnp.int32)
    cvec = cv.at[want].get(mode="promise_in_bounds")

    plsc.subcore_barrier()

    def _ld_block(b, slot):
        bs = pl.ds(slot * _BLKE, _BLKE)
        pltpu.async_copy(src_hbm.at[s, b], srcB.at[bs], bsem)
        pltpu.async_copy(dst_hbm.at[s, b], dstB.at[bs], bsem)
        pltpu.async_copy(val_hbm.at[s, b], valB.at[bs], bsem)

    def _wait_block():
        bs = pl.ds(0, _BLKE)
        pltpu.make_async_copy(src_hbm.at[s, 0], srcB.at[bs], bsem).wait()
        pltpu.make_async_copy(dst_hbm.at[s, 0], dstB.at[bs], bsem).wait()
        pltpu.make_async_copy(val_hbm.at[s, 0], valB.at[bs], bsem).wait()

    def _copy_src(slot, u, w):
        for t in range(_GCH // _LANES):
            srcw[w][pl.ds(t * _LANES, _LANES)] = (
                srcB[pl.ds(slot * _BLKE + u * _GCH + t * _LANES, _LANES)])

    def _gather(w, buf, sem):
        pltpu.async_copy(inp_hbm.at[srcw[w]], buf, sem)

    def _wait_g(buf, sem):
        pltpu.make_async_copy(inp_hbm.at[srcw[0]], buf, sem).wait()

    def _scatter(h, buf, sem):
        pltpu.async_copy(buf, acc_sp.at[dstb[h]], sem, add=True)

    def _wait_s(buf, sem):
        pltpu.make_async_copy(buf, acc_sp.at[dstb[0]], sem).wait()

    _MASK = jnp.full((_LANES,), -65536, jnp.int32)

    def _scale_half(gbuf, sbuf, mslot, u, h):
        base = u * _GCH + h * _SCH

        def _gg(g, carry):
            w16 = valB[pl.ds(mslot * _BLKE + base + g * _LANES, _LANES)] * cvec
            for l in range(_LANES):
                w = w16[l]
                e = h * _SCH + g * _LANES + l
                eo = g * _LANES + l
                for j in range(_DW // _LANES):
                    x = gbuf[e, pl.ds(j * _LANES, _LANES)]
                    lo = plsc.bitcast(x << 16, jnp.float32)
                    hi = plsc.bitcast(x & _MASK, jnp.float32)
                    sbuf[eo, pl.ds(j * 2 * _LANES, _LANES)] = lo * w
                    sbuf[eo, pl.ds((j * 2 + 1) * _LANES, _LANES)] = hi * w
            return carry
        lax.fori_loop(0, _SCH // _LANES, _gg, 0)

    def _copy_dst(mslot, u, h):
        base = u * _GCH + h * _SCH
        for t in range(_SCH // _LANES):
            dstb[h][pl.ds(t * _LANES, _LANES)] = (
                dstB[pl.ds(mslot * _BLKE + base + t * _LANES, _LANES)])

    z32 = jnp.int32(0)
    bs0 = pl.ds(0, _BLKE)
    pltpu.sync_copy(src_hbm.at[s, 0], srcB.at[bs0])
    pltpu.sync_copy(dst_hbm.at[s, 0], dstB.at[bs0])
    pltpu.sync_copy(val_hbm.at[s, 0], valB.at[bs0])
    bs1 = pl.ds(_BLKE, _BLKE)
    pltpu.sync_copy(src_hbm.at[s, 1], srcB.at[bs1])
    pltpu.sync_copy(dst_hbm.at[s, 1], dstB.at[bs1])
    pltpu.sync_copy(val_hbm.at[s, 1], valB.at[bs1])
    _copy_src(z32, 0, 0)
    _copy_src(z32, 1, 1)
    _copy_src(jnp.int32(1), 0, 2)
    for w in range(3):
        _gather(w, gbufs[w], gsems[w])
    _ld_block(jnp.int32(2), 2)
    _scatter(0, sb0, ss0)
    _scatter(1, sb1, ss1)

    _NG = _NBLK * 2

    def _outer(i, carry):
        g0i = 3 * i
        for w in range(3):
            g = g0i + w
            mdyn = g // 2
            mslot = lax.rem(mdyn, 3)
            u = lax.rem(g, 2)
            _wait_g(gbufs[w], gsems[w])
            for h in range(2):
                _wait_s(sbufs[h], ssems[h])
                _copy_dst(mslot, u, h)
                _scale_half(gbufs[w], sbufs[h], mslot, u, h)
                _scatter(h, sbufs[h], ssems[h])

            @pl.when(jnp.logical_and(u == 0, mdyn >= 1))
            def _():
                _ld_block(jnp.minimum(mdyn + 2, _NBLK - 1),
                          lax.rem(mdyn + 2, 3))

            @pl.when(u == 1)
            def _():
                _wait_block()

            gn = jnp.minimum(g + 3, _NG - 1)
            _copy_src(lax.rem(gn // 2, 3), lax.rem(gn, 2), w)
            _gather(w, gbufs[w], gsems[w])
        return carry
    lax.fori_loop(0, _NG // 3, _outer, 0)

    for w in range(3):
        _wait_g(gbufs[w], gsems[w])
    _wait_s(sb0, ss0)
    _wait_s(sb1, ss1)

    plsc.subcore_barrier()
    rsl = pl.ds(s * _ROWS_PT, _ROWS_PT)
    pltpu.sync_copy(acc_sp.at[rsl], acc_hbm.at[c, rsl])


@jax.jit
def _sc_call(inp_i32, src, dst, val, coeff_flat):
    mesh = plsc.VectorSubcoreMesh(core_axis_name="c", subcore_axis_name="s",
                                  num_cores=_NC, num_subcores=_NS)
    return pl.kernel(
        _sc_body,
        out_type=jax.ShapeDtypeStruct((_NB, _NPAD, _D), jnp.float32),
        mesh=mesh,
        compiler_params=pltpu.CompilerParams(needs_layout_passes=False,
                                             use_tc_tiling_on_sc=False),
        scratch_types=[
            pltpu.VMEM_SHARED((_NPAD, _D), jnp.float32),
            pltpu.VMEM((3 * _BLKE,), jnp.int32),
            pltpu.VMEM((3 * _BLKE,), jnp.int32),
            pltpu.VMEM((3 * _BLKE,), jnp.float32),
            pltpu.VMEM((_GCH,), jnp.int32),
            pltpu.VMEM((_GCH,), jnp.int32),
            pltpu.VMEM((_GCH,), jnp.int32),
            pltpu.VMEM((_SCH,), jnp.int32),
            pltpu.VMEM((_SCH,), jnp.int32),
            pltpu.VMEM((_GCH, _DW), jnp.int32),
            pltpu.VMEM((_GCH, _DW), jnp.int32),
            pltpu.VMEM((_GCH, _DW), jnp.int32),
            pltpu.VMEM((_SCH, _D), jnp.float32),
            pltpu.VMEM((_SCH, _D), jnp.float32),
            pltpu.VMEM((_LANES,), jnp.float32),
            pltpu.SemaphoreType.DMA,
            pltpu.SemaphoreType.DMA,
            pltpu.SemaphoreType.DMA,
            pltpu.SemaphoreType.DMA,
            pltpu.SemaphoreType.DMA,
            pltpu.SemaphoreType.DMA,
        ],
    )(inp_i32, src, dst, val, coeff_flat)


_BLK = 2000


def _tc_body(acc_ref, bw_ref, bias_ref, out_ref):
    a0 = acc_ref[0]
    a1 = acc_ref[1]
    out = jnp.dot(a0, bw_ref[0], preferred_element_type=jnp.float32)
    out = out + jnp.dot(a1, bw_ref[1], preferred_element_type=jnp.float32)
    out_ref[...] = out + jnp.sum(bias_ref[...], axis=0)[None, :]


@jax.jit
def _tc_call(acc, basis_weights, bias):
    return pl.pallas_call(
        _tc_body,
        out_shape=jax.ShapeDtypeStruct((_N, _D), jnp.float32),
        grid=(_N // _BLK,),
        in_specs=[
            pl.BlockSpec((_NB, _BLK, _D), lambda i: (0, i, 0)),
            pl.BlockSpec((_NB, _D, _D), lambda i: (0, 0, 0)),
            pl.BlockSpec((_R, _D), lambda i: (0, 0)),
        ],
        out_specs=pl.BlockSpec((_BLK, _D), lambda i: (i, 0)),
    )(acc, basis_weights, bias)


def _edges3(x):
    xp = jnp.pad(x, ((0, 0), (0, _EPAD - _E)))
    return xp.reshape(_NS, _NBLK, _BLKE)


_PERM = np.empty(_D, np.int32)
for _j in range(_D // 32):
    for _t in range(16):
        _PERM[32 * _j + _t] = 32 * _j + 2 * _t
        _PERM[32 * _j + 16 + _t] = 32 * _j + 2 * _t + 1


def kernel(inp, edge_index, edge_val, basis_weights, basis_coeff, bias):
    inp_i32 = jax.lax.bitcast_convert_type(
        inp.astype(jnp.bfloat16).reshape(_N, _DW, 2), jnp.int32)
    dst = _edges3(edge_index[:, 0, :])
    src = _edges3(edge_index[:, 1, :])
    val = _edges3(edge_val)
    coeff_flat = jnp.zeros((_LANES,), jnp.float32).at[: _R * _NB].set(
        basis_coeff.reshape(-1))
    acc = _sc_call(inp_i32, src, dst, val, coeff_flat)
    bw_perm = basis_weights[:, jnp.asarray(_PERM), :]
    return _tc_call(acc, bw_perm, bias)
